# Initial kernel scaffold; baseline (speedup 1.0000x reference)
import jax
import jax.numpy as jnp
from jax.experimental import pallas as pl


def kernel(x, Wg, W1, W2):
    raise NotImplementedError("write your pallas kernel here")


# speedup vs baseline: 1.4769x; 1.4769x over previous
import jax
import jax.numpy as jnp
from jax.experimental import pallas as pl
from jax.experimental.pallas import tpu as pltpu

_TB = 512


def _moe_kernel(x_ref, wg_ref, w1_ref, w2_ref, out_ref, comb_ref, acc_ref):
    e = pl.program_id(0)
    t = pl.program_id(1)
    n_e = pl.num_programs(0)
    TB = x_ref.shape[0]
    E = wg_ref.shape[1]

    row = pl.ds(t * TB, TB)

    @pl.when(e == 0)
    def _gate():
        logits = jnp.dot(x_ref[...], wg_ref[...],
                         preferred_element_type=jnp.float32)
        cols = jax.lax.broadcasted_iota(jnp.int32, (TB, E), 1)
        m1 = jnp.max(logits, axis=-1, keepdims=True)
        i1 = jnp.argmax(logits, axis=-1)
        is1 = cols == i1[:, None]
        masked = jnp.where(is1, -jnp.inf, logits)
        m2 = jnp.max(masked, axis=-1, keepdims=True)
        i2 = jnp.argmax(masked, axis=-1)
        is2 = cols == i2[:, None]
        z = jnp.exp(m2 - m1)
        w_hi = 1.0 / (1.0 + z)
        w_lo = z / (1.0 + z)
        comb_ref[row, :] = jnp.where(is1, w_hi, 0.0) + jnp.where(is2, w_lo, 0.0)

    h = jnp.maximum(
        jnp.dot(x_ref[...], w1_ref[0], preferred_element_type=jnp.float32), 0.0)
    y = jnp.dot(h, w2_ref[0], preferred_element_type=jnp.float32)
    cols = jax.lax.broadcasted_iota(jnp.int32, (TB, E), 1)
    scale = jnp.sum(jnp.where(cols == e, comb_ref[row, :], 0.0), axis=-1,
                    keepdims=True)
    contrib = y * scale

    @pl.when(e == 0)
    def _init():
        acc_ref[row, :] = contrib

    @pl.when(jnp.logical_and(e > 0, e < n_e - 1))
    def _acc():
        acc_ref[row, :] += contrib

    @pl.when(e == n_e - 1)
    def _flush():
        out_ref[...] = acc_ref[row, :] + contrib


def kernel(x, Wg, W1, W2):
    B, T, C = x.shape
    N = B * T
    E = Wg.shape[1]
    DFF = W1.shape[2]
    NT = N // _TB
    xf = x.reshape(N, C)

    out = pl.pallas_call(
        _moe_kernel,
        grid=(E, NT),
        in_specs=[
            pl.BlockSpec((_TB, C), lambda e, t: (t, 0)),
            pl.BlockSpec((C, E), lambda e, t: (0, 0)),
            pl.BlockSpec((1, C, DFF), lambda e, t: (e, 0, 0)),
            pl.BlockSpec((1, DFF, C), lambda e, t: (e, 0, 0)),
        ],
        out_specs=pl.BlockSpec((_TB, C), lambda e, t: (t, 0)),
        out_shape=jax.ShapeDtypeStruct((N, C), jnp.float32),
        scratch_shapes=[
            pltpu.VMEM((N, E), jnp.float32),
            pltpu.VMEM((N, C), jnp.float32),
        ],
    )(xf, Wg, W1, W2)
    return out.reshape(B, T, C)
